# baseline (speedup 1.0000x reference)
import jax
import jax.numpy as jnp
from jax import lax
from jax.experimental import pallas as pl
from jax.experimental.pallas import tpu as pltpu

NF = 8
NC = 8
PHASE = NF + NC


def _linear_kernel(x_ref, w_ref, b_ref, o_ref, ws_ref):
    t = pl.program_id(0)
    u = t % PHASE
    rs = w_ref.shape[0]

    @pl.when(u < NF)
    def _fill():
        ws_ref[pl.ds(u * rs, rs), :] = w_ref[...].astype(jnp.bfloat16)

    @pl.when(u >= NF)
    def _compute():
        xb = x_ref[...].astype(jnp.bfloat16)
        o_ref[...] = lax.dot_general(
            xb, ws_ref[...], (((1,), (1,)), ((), ())),
            preferred_element_type=jnp.float32,
        ) + b_ref[...]


def kernel(input, weight, bias):
    m, kdim = input.shape
    n, _ = weight.shape
    bias2 = bias.reshape(1, n)
    nh = n // 2
    rs = nh // NF
    bm = m // NC
    return pl.pallas_call(
        _linear_kernel,
        grid=(2 * PHASE,),
        in_specs=[
            pl.BlockSpec(
                (bm, kdim), lambda t: (jnp.maximum(t % PHASE - NF, 0), 0)
            ),
            pl.BlockSpec(
                (rs, kdim),
                lambda t: ((t // PHASE) * NF + jnp.minimum(t % PHASE, NF - 1), 0),
            ),
            pl.BlockSpec((1, nh), lambda t: (0, t // PHASE)),
        ],
        out_specs=pl.BlockSpec(
            (bm, nh),
            lambda t: (jnp.maximum(t % PHASE - NF, 0), t // PHASE),
        ),
        out_shape=jax.ShapeDtypeStruct((m, n), jnp.float32),
        scratch_shapes=[pltpu.VMEM((n // 2, kdim), jnp.bfloat16)],
        compiler_params=pltpu.CompilerParams(
            dimension_semantics=("arbitrary",),
        ),
    )(input, weight, bias2)


# speedup vs baseline: 1.0230x; 1.0230x over previous
import jax
import jax.numpy as jnp
from jax import lax
from jax.experimental import pallas as pl
from jax.experimental.pallas import tpu as pltpu

FILL = 16
BM = 256


def _linear_kernel(x_ref, w_ref, b_ref, o_ref, ws_ref):
    t = pl.program_id(0)
    rs = w_ref.shape[0]

    @pl.when(t < FILL)
    def _fill():
        ws_ref[pl.ds(t * rs, rs), :] = w_ref[...].astype(jnp.bfloat16)

    @pl.when(t >= FILL)
    def _compute():
        xb = x_ref[...].astype(jnp.bfloat16)
        o_ref[...] = lax.dot_general(
            xb, ws_ref[...], (((1,), (1,)), ((), ())),
            preferred_element_type=jnp.float32,
        ) + b_ref[...]


def kernel(input, weight, bias):
    m, kdim = input.shape
    n, _ = weight.shape
    bias2 = bias.reshape(1, n)
    nsteps = FILL + m // BM
    return pl.pallas_call(
        _linear_kernel,
        grid=(nsteps,),
        in_specs=[
            pl.BlockSpec((BM, kdim), lambda t: (jnp.maximum(t - FILL, 0), 0)),
            pl.BlockSpec((n // FILL, kdim), lambda t: (jnp.minimum(t, FILL - 1), 0)),
            pl.BlockSpec((1, n), lambda t: (0, 0)),
        ],
        out_specs=pl.BlockSpec((BM, n), lambda t: (jnp.maximum(t - FILL, 0), 0)),
        out_shape=jax.ShapeDtypeStruct((m, n), jnp.float32),
        scratch_shapes=[pltpu.VMEM((n, kdim), jnp.bfloat16)],
        compiler_params=pltpu.CompilerParams(
            dimension_semantics=("arbitrary",),
        ),
    )(input, weight, bias2)
